# baseline (speedup 1.0000x reference)
import functools
from math import sqrt

import jax
import jax.numpy as jnp
import numpy as np
from jax import lax
from jax.experimental import pallas as pl
from jax.experimental.pallas import tpu as pltpu
from jax.experimental.pallas import tpu_sc as plsc

_B, _L, _H, _D = 2, 4096, 16, 64
_NBH = _B * _H
_U = 45
_SP = 48
_NL = 16
_ROWS = _NL * _SP
_NRI = _ROWS // 128
_NCH = _L // _NL
_IGRP = 8
_IROWS = _IGRP * _NRI
_BQ = 256
_NQB = _L // _BQ


def _sc_gather(kflat, tab):
    mesh = plsc.VectorSubcoreMesh(core_axis_name="c", subcore_axis_name="s")
    out_type = jax.ShapeDtypeStruct((_NBH, _L * _SP, _D), jnp.float32)
    scratch = [
        pltpu.VMEM((_ROWS, _D), jnp.float32),
        pltpu.VMEM((_ROWS, _D), jnp.float32),
        pltpu.VMEM((_IROWS, 128), jnp.int32),
        pltpu.SemaphoreType.DMA,
        pltpu.SemaphoreType.DMA,
        pltpu.SemaphoreType.DMA,
        pltpu.SemaphoreType.DMA,
    ]

    @functools.partial(pl.kernel, mesh=mesh, out_type=out_type,
                       scratch_types=scratch,
                       compiler_params=pltpu.CompilerParams(
                           use_tc_tiling_on_sc=False))
    def run(k_hbm, t_hbm, kg_hbm, kg0, kg1, idxb, g0, g1, w0, w1):
        bh = lax.axis_index("s") * 2 + lax.axis_index("c")
        kgs = (kg0, kg1)
        gsems = (g0, g1)
        wsems = (w0, w1)

        def stage_idx(g):
            pltpu.sync_copy(t_hbm.at[bh, pl.ds(g * _IROWS, _IROWS)], idxb)

        def gdesc(c, p, r):
            lrow = (c % _IGRP) * _NRI + r
            return pltpu.make_async_copy(
                k_hbm.at[idxb.at[lrow]],
                kgs[p].at[pl.ds(r * 128, 128)], gsems[p])

        def wdesc(c, p):
            return pltpu.make_async_copy(
                kgs[p], kg_hbm.at[bh, pl.ds(c * _ROWS, _ROWS)], wsems[p])

        def start_g(c, p):
            for r in range(_NRI):
                gdesc(c, p, r).start()

        def wait_g(c, p):
            for r in range(_NRI):
                gdesc(c, p, r).wait()

        stage_idx(0)
        start_g(0, 0)

        def two(j, carry):
            for p in (0, 1):
                c = j * 2 + p
                wait_g(c, p)
                wdesc(c, p).start()
                if p == 0:
                    @pl.when(c > 0)
                    def _():
                        wdesc(c - 1, 1).wait()

                    @pl.when(c + 1 < _NCH)
                    def _():
                        start_g(c + 1, 1)
                else:
                    wdesc(c - 1, 0).wait()

                    @pl.when(c + 1 < _NCH)
                    def _():
                        @pl.when((c + 1) % _IGRP == 0)
                        def _():
                            stage_idx((c + 1) // _IGRP)
                        start_g(c + 1, 0)
            return carry

        lax.fori_loop(0, _NCH // 2, two, 0)
        wdesc(_NCH - 1, 1).wait()

    return run(kflat, tab)


_QSEG = [(0, 2), (2, 5), (5, 8), (8, 10), (10, 13), (13, 16)]


def _sc_m_scores(kflat, q3, tab):
    mesh = plsc.VectorSubcoreMesh(core_axis_name="c", subcore_axis_name="s")
    out_type = (
        jax.ShapeDtypeStruct((_NBH, _L), jnp.float32),
        jax.ShapeDtypeStruct((_NBH, _L), jnp.float32),
    )
    scratch = [
        pltpu.VMEM((_ROWS, _D), jnp.float32),
        pltpu.VMEM((_NL * _IGRP, _D), jnp.float32),
        pltpu.VMEM((_IROWS, 128), jnp.int32),
        pltpu.VMEM((_L,), jnp.float32),
        pltpu.VMEM((_L,), jnp.float32),
    ] + [pltpu.SemaphoreType.DMA] * _NRI

    @functools.partial(pl.kernel, mesh=mesh, out_type=out_type,
                       scratch_types=scratch,
                       compiler_params=pltpu.CompilerParams(
                           use_tc_tiling_on_sc=False,
                           needs_layout_passes=False))
    def run(k_hbm, q_hbm, t_hbm, mx_hbm, ms_hbm,
            kg, qgb, idxb, mxr, msr, *sems):
        bh = lax.axis_index("s") * 2 + lax.axis_index("c")
        iota16 = lax.iota(jnp.int32, 16)

        def desc(c, r):
            lrow = (c % _IGRP) * _NRI + r
            return pltpu.make_async_copy(
                k_hbm.at[idxb.at[lrow]],
                kg.at[pl.ds(r * 128, 128)], sems[r])

        def gbody(g, carry):
            pltpu.sync_copy(t_hbm.at[bh, pl.ds(g * _IROWS, _IROWS)], idxb)
            pltpu.sync_copy(
                q_hbm.at[bh, pl.ds(g * (_NL * _IGRP), _NL * _IGRP)], qgb)

            def cbody(c, carry):
                for r in range(_NRI):
                    desc(c, r).start()
                qoff = (c % _IGRP) * _NL
                vmax = jnp.zeros((16,), jnp.float32)
                vsum = jnp.zeros((16,), jnp.float32)

                def lbody(i, carry):
                    vmax, vsum = carry
                    qrow = qoff + i
                    q0 = qgb[qrow, pl.ds(0, 16)]
                    q1 = qgb[qrow, pl.ds(16, 16)]
                    q2 = qgb[qrow, pl.ds(32, 16)]
                    q3v = qgb[qrow, pl.ds(48, 16)]
                    smax = jnp.float32(-3.0e38)
                    ssum = jnp.float32(0.0)
                    for s in range(_U):
                        row = i * _SP + s
                        a = ((kg[row, pl.ds(0, 16)] * q0
                              + kg[row, pl.ds(16, 16)] * q1)
                             + (kg[row, pl.ds(32, 16)] * q2
                                + kg[row, pl.ds(48, 16)] * q3v))
                        dot = plsc.cumsum(a)[15]
                        smax = jnp.maximum(smax, dot)
                        ssum = ssum + dot
                    lane = iota16 == i
                    return (jnp.where(lane, smax, vmax),
                            jnp.where(lane, ssum, vsum))

                for r, (lo, hi) in enumerate(_QSEG):
                    desc(c, r).wait()
                    vmax, vsum = lax.fori_loop(lo, hi, lbody, (vmax, vsum))
                mxr[pl.ds(c * _NL, _NL)] = vmax
                msr[pl.ds(c * _NL, _NL)] = vsum
                return carry

            lax.fori_loop(g * _IGRP, (g + 1) * _IGRP, cbody, 0)
            return carry

        lax.fori_loop(0, _NCH // _IGRP, gbody, 0)
        pltpu.sync_copy(mxr, mx_hbm.at[bh])
        pltpu.sync_copy(msr, ms_hbm.at[bh])

    return run(kflat, q3, tab)


def _sc_m_scores2(kflat, qt3, tab2):
    mesh = plsc.VectorSubcoreMesh(core_axis_name="c", subcore_axis_name="s")
    out_type = (
        jax.ShapeDtypeStruct((_NBH, _L), jnp.float32),
        jax.ShapeDtypeStruct((_NBH, _L), jnp.float32),
    )
    scratch = [
        pltpu.VMEM((_ROWS, _D), jnp.float32),
        pltpu.VMEM((_D, _NL * _IGRP), jnp.float32),
        pltpu.VMEM((_IROWS, 128), jnp.int32),
        pltpu.VMEM((_L,), jnp.float32),
        pltpu.VMEM((_L,), jnp.float32),
    ] + [pltpu.SemaphoreType.DMA] * _NRI

    @functools.partial(pl.kernel, mesh=mesh, out_type=out_type,
                       scratch_types=scratch,
                       compiler_params=pltpu.CompilerParams(
                           use_tc_tiling_on_sc=False,
                           needs_layout_passes=False))
    def run(k_hbm, q_hbm, t_hbm, mx_hbm, ms_hbm,
            kg, qtb, idxb, mxr, msr, *sems):
        bh = lax.axis_index("s") * 2 + lax.axis_index("c")
        iota16 = lax.iota(jnp.int32, 16)

        def desc(c, r):
            lrow = (c % _IGRP) * _NRI + r
            return pltpu.make_async_copy(
                k_hbm.at[idxb.at[lrow]],
                kg.at[pl.ds(r * 128, 128)], sems[r])

        def gbody(g, carry):
            pltpu.sync_copy(t_hbm.at[bh, pl.ds(g * _IROWS, _IROWS)], idxb)
            pltpu.sync_copy(
                q_hbm.at[bh, :, pl.ds(g * (_NL * _IGRP), _NL * _IGRP)], qtb)

            def cbody(c, carry):
                for r in range(_NRI):
                    desc(c, r).start()
                qoff = (c % _IGRP) * _NL
                vmax = jnp.full((16,), -3.0e38, jnp.float32)
                vsum = jnp.zeros((16,), jnp.float32)
                for r in range(_NRI):
                    desc(c, r).wait()
                    sg = r * 8
                    nk = min(8, _U - sg)
                    rows = [iota16 + (sg + k) * _NL for k in range(nk)]

                    def dbody(dd, accs):
                        qd = qtb[dd, pl.ds(qoff, _NL)]
                        dcol = jnp.full((16,), dd, jnp.int32)
                        return tuple(
                            acc + plsc.load_gather(kg, [rows[k], dcol]) * qd
                            for k, acc in enumerate(accs)
                        )

                    accs = lax.fori_loop(
                        0, _D, dbody,
                        tuple(jnp.zeros((16,), jnp.float32)
                              for _ in range(nk)),
                        unroll=8)
                    for acc in accs:
                        vmax = jnp.maximum(vmax, acc)
                        vsum = vsum + acc
                mxr[pl.ds(c * _NL, _NL)] = vmax
                msr[pl.ds(c * _NL, _NL)] = vsum
                return carry

            lax.fori_loop(g * _IGRP, (g + 1) * _IGRP, cbody, 0)
            return carry

        lax.fori_loop(0, _NCH // _IGRP, gbody, 0)
        pltpu.sync_copy(mxr, mx_hbm.at[bh])
        pltpu.sync_copy(msr, ms_hbm.at[bh])

    return run(kflat, qt3, tab2)


def _tc_mreduce_body(kg_ref, q_ref, mx_ref, ms_ref):
    kg = kg_ref[0, 0].reshape(_BQ, _SP, _D)
    q = q_ref[0].reshape(_BQ, 1, _D)
    dots = jnp.sum(kg * q, axis=-1)
    svalid = lax.broadcasted_iota(jnp.int32, (_BQ, _SP), 1) < _U
    mx_ref[0, 0, 0] = jnp.max(jnp.where(svalid, dots, jnp.float32(-3.0e38)),
                              axis=-1)
    ms_ref[0, 0, 0] = jnp.sum(jnp.where(svalid, dots, jnp.float32(0.0)),
                              axis=-1)


def _tc_mreduce(kg, q3, interpret=False):
    mx4, ms4 = pl.pallas_call(
        _tc_mreduce_body,
        grid=(_NBH, _NQB),
        in_specs=[
            pl.BlockSpec((1, 1, _BQ * _SP, _D), lambda i, j: (i, j, 0, 0)),
            pl.BlockSpec((1, _BQ, _D), lambda i, j: (i, j, 0)),
        ],
        out_specs=[
            pl.BlockSpec((1, 1, 1, _BQ), lambda i, j: (i, j, 0, 0)),
            pl.BlockSpec((1, 1, 1, _BQ), lambda i, j: (i, j, 0, 0)),
        ],
        out_shape=[
            jax.ShapeDtypeStruct((_NBH, _NQB, 1, _BQ), jnp.float32),
            jax.ShapeDtypeStruct((_NBH, _NQB, 1, _BQ), jnp.float32),
        ],
        interpret=interpret,
    )(kg, q3)
    return mx4.reshape(_NBH, _L), ms4.reshape(_NBH, _L)


def _tc_topk_body(mx_ref, ms_ref, o_ref):
    M = mx_ref[...] - ms_ref[...] * jnp.float32(1.0 / _L)
    ci = lax.broadcasted_iota(jnp.int32, (_NBH, _L), 1).astype(jnp.float32)
    lane = lax.broadcasted_iota(jnp.int32, (_NBH, 128), 1).astype(jnp.float32)
    out = jnp.zeros((_NBH, 128), jnp.float32)
    X = M
    for i in range(_U):
        rmax = jnp.max(X, axis=1, keepdims=True)
        loc = jnp.min(jnp.where(X == rmax, ci, jnp.float32(1e9)),
                      axis=1, keepdims=True)
        out = jnp.where(lane == jnp.float32(i), loc, out)
        X = jnp.where(ci == loc, jnp.float32(-3.0e38), X)
    o_ref[...] = out.astype(jnp.int32)


def _tc_topk(mmax, msum, interpret=False):
    return pl.pallas_call(
        _tc_topk_body,
        out_shape=jax.ShapeDtypeStruct((_NBH, 128), jnp.int32),
        interpret=interpret,
    )(mmax, msum)


def _tc_attend_body(mt_ref, q_ref, k_ref, v_ref, o_ref, scr):
    for u in range(_U):
        r = mt_ref[0, 0, u]
        scr[pl.ds(u, 1), :] = q_ref[0, pl.ds(r, 1), :]
    scr[pl.ds(_U, 3), :] = jnp.zeros((3, _D), jnp.float32)
    qr = scr[...]
    k = k_ref[0]
    v = v_ref[0]
    S = lax.dot_general(qr, k, (((1,), (1,)), ((), ())),
                        preferred_element_type=jnp.float32)
    S = S * jnp.float32(1.0 / sqrt(_D))
    smx = jnp.max(S, axis=1, keepdims=True)
    E = jnp.exp(S - smx)
    P = E / jnp.sum(E, axis=1, keepdims=True)
    U = lax.dot_general(P, v, (((1,), (0,)), ((), ())),
                        preferred_element_type=jnp.float32)
    vmean = jnp.mean(v, axis=0, keepdims=True)
    o_ref[0] = jnp.broadcast_to(vmean, (_L, _D))
    for u in range(_U):
        r = mt_ref[0, 0, u]
        o_ref[0, pl.ds(r, 1), :] = U[u:u + 1, :]


def _tc_attend(mtop, q3, k3, v3, interpret=False):
    return pl.pallas_call(
        _tc_attend_body,
        grid=(_NBH,),
        in_specs=[
            pl.BlockSpec((1, 1, 128), lambda i: (i, 0, 0),
                         memory_space=pltpu.SMEM),
            pl.BlockSpec((1, _L, _D), lambda i: (i, 0, 0)),
            pl.BlockSpec((1, _L, _D), lambda i: (i, 0, 0)),
            pl.BlockSpec((1, _L, _D), lambda i: (i, 0, 0)),
        ],
        out_specs=pl.BlockSpec((1, _L, _D), lambda i: (i, 0, 0)),
        out_shape=jax.ShapeDtypeStruct((_NBH, _L, _D), jnp.float32),
        scratch_shapes=[pltpu.VMEM((_SP, _D), jnp.float32)],
        interpret=interpret,
    )(mtop.reshape(_NBH, 1, 128), q3, k3, v3)


def _tc_prep_body(k_ref, q_ref, kf_ref, qt_ref):
    kf_ref[...] = k_ref[0]
    qt_ref[0] = jnp.transpose(q_ref[0], (1, 0))


def _tc_prep(k3, q3, interpret=False):
    return pl.pallas_call(
        _tc_prep_body,
        grid=(_NBH,),
        in_specs=[pl.BlockSpec((1, _L, _D), lambda i: (i, 0, 0)),
                  pl.BlockSpec((1, _L, _D), lambda i: (i, 0, 0))],
        out_specs=[pl.BlockSpec((_L, _D), lambda i: (i, 0)),
                   pl.BlockSpec((1, _D, _L), lambda i: (i, 0, 0))],
        out_shape=[jax.ShapeDtypeStruct((_NBH * _L, _D), jnp.float32),
                   jax.ShapeDtypeStruct((_NBH, _D, _L), jnp.float32)],
        interpret=interpret,
    )(k3, q3)


def _sample_table():
    idx = jax.random.randint(jax.random.key(42), (_L, _U), 0, _L)
    idx48 = jnp.concatenate([idx, jnp.tile(idx[:, _U - 1:_U], (1, _SP - _U))],
                            axis=1)
    flat = (idx48.reshape(_NCH, _NL, _SP)
            .transpose(0, 2, 1)
            .reshape(-1).astype(jnp.int32))
    tab = flat[None, :] + (jnp.arange(_NBH, dtype=jnp.int32) * _L)[:, None]
    return tab.reshape(_NBH, (_L * _SP) // 128, 128)


def kernel(queries, keys, values):
    q3 = jnp.transpose(queries, (0, 2, 1, 3)).reshape(_NBH, _L, _D)
    k3 = jnp.transpose(keys, (0, 2, 1, 3)).reshape(_NBH, _L, _D)
    v3 = jnp.transpose(values, (0, 2, 1, 3)).reshape(_NBH, _L, _D)
    kflat, qt3 = _tc_prep(k3, q3)
    tab = _sample_table()
    mmax, msum = _sc_m_scores2(kflat, qt3, tab)
    mtop = _tc_topk(mmax, msum)
    ctx = _tc_attend(mtop, q3, k3, v3)
    return jnp.transpose(ctx.reshape(_B, _H, _L, _D), (0, 2, 1, 3))


# speedup vs baseline: 2.3883x; 2.3883x over previous
import functools
from math import sqrt

import jax
import jax.numpy as jnp
import numpy as np
from jax import lax
from jax.experimental import pallas as pl
from jax.experimental.pallas import tpu as pltpu
from jax.experimental.pallas import tpu_sc as plsc

_B, _L, _H, _D = 2, 4096, 16, 64
_NBH = _B * _H
_U = 45
_SP = 48
_NL = 16
_ROWS = _NL * _SP
_NRI = _ROWS // 128
_NCH = _L // _NL
_IGRP = 8
_IROWS = _IGRP * _NRI
_BQ = 256
_NQB = _L // _BQ


def _sc_gather(kflat, tab):
    mesh = plsc.VectorSubcoreMesh(core_axis_name="c", subcore_axis_name="s")
    out_type = jax.ShapeDtypeStruct((_NBH, _L * _SP, _D), jnp.float32)
    scratch = [
        pltpu.VMEM((_ROWS, _D), jnp.float32),
        pltpu.VMEM((_ROWS, _D), jnp.float32),
        pltpu.VMEM((_IROWS, 128), jnp.int32),
        pltpu.SemaphoreType.DMA,
        pltpu.SemaphoreType.DMA,
        pltpu.SemaphoreType.DMA,
        pltpu.SemaphoreType.DMA,
    ]

    @functools.partial(pl.kernel, mesh=mesh, out_type=out_type,
                       scratch_types=scratch,
                       compiler_params=pltpu.CompilerParams(
                           use_tc_tiling_on_sc=False))
    def run(k_hbm, t_hbm, kg_hbm, kg0, kg1, idxb, g0, g1, w0, w1):
        bh = lax.axis_index("s") * 2 + lax.axis_index("c")
        kgs = (kg0, kg1)
        gsems = (g0, g1)
        wsems = (w0, w1)

        def stage_idx(g):
            pltpu.sync_copy(t_hbm.at[bh, pl.ds(g * _IROWS, _IROWS)], idxb)

        def gdesc(c, p, r):
            lrow = (c % _IGRP) * _NRI + r
            return pltpu.make_async_copy(
                k_hbm.at[idxb.at[lrow]],
                kgs[p].at[pl.ds(r * 128, 128)], gsems[p])

        def wdesc(c, p):
            return pltpu.make_async_copy(
                kgs[p], kg_hbm.at[bh, pl.ds(c * _ROWS, _ROWS)], wsems[p])

        def start_g(c, p):
            for r in range(_NRI):
                gdesc(c, p, r).start()

        def wait_g(c, p):
            for r in range(_NRI):
                gdesc(c, p, r).wait()

        stage_idx(0)
        start_g(0, 0)

        def two(j, carry):
            for p in (0, 1):
                c = j * 2 + p
                wait_g(c, p)
                wdesc(c, p).start()
                if p == 0:
                    @pl.when(c > 0)
                    def _():
                        wdesc(c - 1, 1).wait()

                    @pl.when(c + 1 < _NCH)
                    def _():
                        start_g(c + 1, 1)
                else:
                    wdesc(c - 1, 0).wait()

                    @pl.when(c + 1 < _NCH)
                    def _():
                        @pl.when((c + 1) % _IGRP == 0)
                        def _():
                            stage_idx((c + 1) // _IGRP)
                        start_g(c + 1, 0)
            return carry

        lax.fori_loop(0, _NCH // 2, two, 0)
        wdesc(_NCH - 1, 1).wait()

    return run(kflat, tab)


_QSEG = [(0, 2), (2, 5), (5, 8), (8, 10), (10, 13), (13, 16)]


def _sc_m_scores(kflat, q3, tab):
    mesh = plsc.VectorSubcoreMesh(core_axis_name="c", subcore_axis_name="s")
    out_type = (
        jax.ShapeDtypeStruct((_NBH, _L), jnp.float32),
        jax.ShapeDtypeStruct((_NBH, _L), jnp.float32),
    )
    scratch = [
        pltpu.VMEM((_ROWS, _D), jnp.float32),
        pltpu.VMEM((_NL * _IGRP, _D), jnp.float32),
        pltpu.VMEM((_IROWS, 128), jnp.int32),
        pltpu.VMEM((_L,), jnp.float32),
        pltpu.VMEM((_L,), jnp.float32),
    ] + [pltpu.SemaphoreType.DMA] * _NRI

    @functools.partial(pl.kernel, mesh=mesh, out_type=out_type,
                       scratch_types=scratch,
                       compiler_params=pltpu.CompilerParams(
                           use_tc_tiling_on_sc=False,
                           needs_layout_passes=False))
    def run(k_hbm, q_hbm, t_hbm, mx_hbm, ms_hbm,
            kg, qgb, idxb, mxr, msr, *sems):
        bh = lax.axis_index("s") * 2 + lax.axis_index("c")
        iota16 = lax.iota(jnp.int32, 16)

        def desc(c, r):
            lrow = (c % _IGRP) * _NRI + r
            return pltpu.make_async_copy(
                k_hbm.at[idxb.at[lrow]],
                kg.at[pl.ds(r * 128, 128)], sems[r])

        def gbody(g, carry):
            pltpu.sync_copy(t_hbm.at[bh, pl.ds(g * _IROWS, _IROWS)], idxb)
            pltpu.sync_copy(
                q_hbm.at[bh, pl.ds(g * (_NL * _IGRP), _NL * _IGRP)], qgb)

            def cbody(c, carry):
                for r in range(_NRI):
                    desc(c, r).start()
                qoff = (c % _IGRP) * _NL
                vmax = jnp.zeros((16,), jnp.float32)
                vsum = jnp.zeros((16,), jnp.float32)

                def lbody(i, carry):
                    vmax, vsum = carry
                    qrow = qoff + i
                    q0 = qgb[qrow, pl.ds(0, 16)]
                    q1 = qgb[qrow, pl.ds(16, 16)]
                    q2 = qgb[qrow, pl.ds(32, 16)]
                    q3v = qgb[qrow, pl.ds(48, 16)]
                    smax = jnp.float32(-3.0e38)
                    ssum = jnp.float32(0.0)
                    for s in range(_U):
                        row = i * _SP + s
                        a = ((kg[row, pl.ds(0, 16)] * q0
                              + kg[row, pl.ds(16, 16)] * q1)
                             + (kg[row, pl.ds(32, 16)] * q2
                                + kg[row, pl.ds(48, 16)] * q3v))
                        dot = plsc.cumsum(a)[15]
                        smax = jnp.maximum(smax, dot)
                        ssum = ssum + dot
                    lane = iota16 == i
                    return (jnp.where(lane, smax, vmax),
                            jnp.where(lane, ssum, vsum))

                for r, (lo, hi) in enumerate(_QSEG):
                    desc(c, r).wait()
                    vmax, vsum = lax.fori_loop(lo, hi, lbody, (vmax, vsum),
                                               unroll=True)
                mxr[pl.ds(c * _NL, _NL)] = vmax
                msr[pl.ds(c * _NL, _NL)] = vsum
                return carry

            lax.fori_loop(g * _IGRP, (g + 1) * _IGRP, cbody, 0)
            return carry

        lax.fori_loop(0, _NCH // _IGRP, gbody, 0)
        pltpu.sync_copy(mxr, mx_hbm.at[bh])
        pltpu.sync_copy(msr, ms_hbm.at[bh])

    return run(kflat, q3, tab)


def _tc_mreduce_body(kg_ref, q_ref, mx_ref, ms_ref):
    kg = kg_ref[0, 0].reshape(_BQ, _SP, _D)
    q = q_ref[0].reshape(_BQ, 1, _D)
    dots = jnp.sum(kg * q, axis=-1)
    svalid = lax.broadcasted_iota(jnp.int32, (_BQ, _SP), 1) < _U
    mx_ref[0, 0, 0] = jnp.max(jnp.where(svalid, dots, jnp.float32(-3.0e38)),
                              axis=-1)
    ms_ref[0, 0, 0] = jnp.sum(jnp.where(svalid, dots, jnp.float32(0.0)),
                              axis=-1)


def _tc_mreduce(kg, q3, interpret=False):
    mx4, ms4 = pl.pallas_call(
        _tc_mreduce_body,
        grid=(_NBH, _NQB),
        in_specs=[
            pl.BlockSpec((1, 1, _BQ * _SP, _D), lambda i, j: (i, j, 0, 0)),
            pl.BlockSpec((1, _BQ, _D), lambda i, j: (i, j, 0)),
        ],
        out_specs=[
            pl.BlockSpec((1, 1, 1, _BQ), lambda i, j: (i, j, 0, 0)),
            pl.BlockSpec((1, 1, 1, _BQ), lambda i, j: (i, j, 0, 0)),
        ],
        out_shape=[
            jax.ShapeDtypeStruct((_NBH, _NQB, 1, _BQ), jnp.float32),
            jax.ShapeDtypeStruct((_NBH, _NQB, 1, _BQ), jnp.float32),
        ],
        interpret=interpret,
    )(kg, q3)
    return mx4.reshape(_NBH, _L), ms4.reshape(_NBH, _L)


def _tc_topk_body(mx_ref, ms_ref, o_ref):
    M = mx_ref[...] - ms_ref[...] * jnp.float32(1.0 / _L)
    ci = lax.broadcasted_iota(jnp.int32, (_NBH, _L), 1).astype(jnp.float32)
    lane = lax.broadcasted_iota(jnp.int32, (_NBH, 128), 1).astype(jnp.float32)
    out = jnp.zeros((_NBH, 128), jnp.float32)
    X = M
    for i in range(_U):
        rmax = jnp.max(X, axis=1, keepdims=True)
        loc = jnp.min(jnp.where(X == rmax, ci, jnp.float32(1e9)),
                      axis=1, keepdims=True)
        out = jnp.where(lane == jnp.float32(i), loc, out)
        X = jnp.where(ci == loc, jnp.float32(-3.0e38), X)
    o_ref[...] = out.astype(jnp.int32)


def _tc_topk(mmax, msum, interpret=False):
    return pl.pallas_call(
        _tc_topk_body,
        out_shape=jax.ShapeDtypeStruct((_NBH, 128), jnp.int32),
        interpret=interpret,
    )(mmax, msum)


def _tc_attend_body(mt_ref, q_ref, k_ref, v_ref, o_ref, scr):
    for u in range(_U):
        r = mt_ref[0, 0, u]
        scr[pl.ds(u, 1), :] = q_ref[0, pl.ds(r, 1), :]
    scr[pl.ds(_U, 3), :] = jnp.zeros((3, _D), jnp.float32)
    qr = scr[...]
    k = k_ref[0]
    v = v_ref[0]
    S = lax.dot_general(qr, k, (((1,), (1,)), ((), ())),
                        preferred_element_type=jnp.float32)
    S = S * jnp.float32(1.0 / sqrt(_D))
    smx = jnp.max(S, axis=1, keepdims=True)
    E = jnp.exp(S - smx)
    P = E / jnp.sum(E, axis=1, keepdims=True)
    U = lax.dot_general(P, v, (((1,), (0,)), ((), ())),
                        preferred_element_type=jnp.float32)
    vmean = jnp.mean(v, axis=0, keepdims=True)
    o_ref[0] = jnp.broadcast_to(vmean, (_L, _D))
    for u in range(_U):
        r = mt_ref[0, 0, u]
        o_ref[0, pl.ds(r, 1), :] = U[u:u + 1, :]


def _tc_attend(mtop, q3, k3, v3, interpret=False):
    return pl.pallas_call(
        _tc_attend_body,
        grid=(_NBH,),
        in_specs=[
            pl.BlockSpec((1, 1, 128), lambda i: (i, 0, 0),
                         memory_space=pltpu.SMEM),
            pl.BlockSpec((1, _L, _D), lambda i: (i, 0, 0)),
            pl.BlockSpec((1, _L, _D), lambda i: (i, 0, 0)),
            pl.BlockSpec((1, _L, _D), lambda i: (i, 0, 0)),
        ],
        out_specs=pl.BlockSpec((1, _L, _D), lambda i: (i, 0, 0)),
        out_shape=jax.ShapeDtypeStruct((_NBH, _L, _D), jnp.float32),
        scratch_shapes=[pltpu.VMEM((_SP, _D), jnp.float32)],
        interpret=interpret,
    )(mtop.reshape(_NBH, 1, 128), q3, k3, v3)


def _tc_prep_body(k_ref, kf_ref):
    kf_ref[...] = k_ref[0]


def _tc_prep(k3, interpret=False):
    return pl.pallas_call(
        _tc_prep_body,
        grid=(_NBH,),
        in_specs=[pl.BlockSpec((1, _L, _D), lambda i: (i, 0, 0))],
        out_specs=pl.BlockSpec((_L, _D), lambda i: (i, 0)),
        out_shape=jax.ShapeDtypeStruct((_NBH * _L, _D), jnp.float32),
        interpret=interpret,
    )(k3)


def _sample_table():
    idx = jax.random.randint(jax.random.key(42), (_L, _U), 0, _L)
    idx48 = jnp.concatenate([idx, jnp.tile(idx[:, _U - 1:_U], (1, _SP - _U))],
                            axis=1)
    flat = idx48.reshape(-1).astype(jnp.int32)
    tab = flat[None, :] + (jnp.arange(_NBH, dtype=jnp.int32) * _L)[:, None]
    return tab.reshape(_NBH, (_L * _SP) // 128, 128)


def kernel(queries, keys, values):
    q3 = jnp.transpose(queries, (0, 2, 1, 3)).reshape(_NBH, _L, _D)
    k3 = jnp.transpose(keys, (0, 2, 1, 3)).reshape(_NBH, _L, _D)
    v3 = jnp.transpose(values, (0, 2, 1, 3)).reshape(_NBH, _L, _D)
    kflat = _tc_prep(k3)
    tab = _sample_table()
    mmax, msum = _sc_m_scores(kflat, q3, tab)
    mtop = _tc_topk(mmax, msum)
    ctx = _tc_attend(mtop, q3, k3, v3)
    return jnp.transpose(ctx.reshape(_B, _H, _L, _D), (0, 2, 1, 3))


# speedup vs baseline: 4.2583x; 1.7830x over previous
import functools
from math import sqrt

import jax
import jax.numpy as jnp
import numpy as np
from jax import lax
from jax.experimental import pallas as pl
from jax.experimental.pallas import tpu as pltpu
from jax.experimental.pallas import tpu_sc as plsc

_B, _L, _H, _D = 2, 4096, 16, 64
_NBH = _B * _H
_U = 45
_SP = 48
_NL = 16
_ROWS = _NL * _SP
_NRI = _ROWS // 128
_NCH = _L // _NL
_IGRP = 8
_IROWS = _IGRP * _NRI
_BQ = 256
_NQB = _L // _BQ


def _sc_gather(kflat, tab):
    mesh = plsc.VectorSubcoreMesh(core_axis_name="c", subcore_axis_name="s")
    out_type = jax.ShapeDtypeStruct((_NBH, _L * _SP, _D), jnp.float32)
    scratch = [
        pltpu.VMEM((_ROWS, _D), jnp.float32),
        pltpu.VMEM((_ROWS, _D), jnp.float32),
        pltpu.VMEM((_IROWS, 128), jnp.int32),
        pltpu.SemaphoreType.DMA,
        pltpu.SemaphoreType.DMA,
        pltpu.SemaphoreType.DMA,
        pltpu.SemaphoreType.DMA,
    ]

    @functools.partial(pl.kernel, mesh=mesh, out_type=out_type,
                       scratch_types=scratch,
                       compiler_params=pltpu.CompilerParams(
                           use_tc_tiling_on_sc=False))
    def run(k_hbm, t_hbm, kg_hbm, kg0, kg1, idxb, g0, g1, w0, w1):
        bh = lax.axis_index("s") * 2 + lax.axis_index("c")
        kgs = (kg0, kg1)
        gsems = (g0, g1)
        wsems = (w0, w1)

        def stage_idx(g):
            pltpu.sync_copy(t_hbm.at[bh, pl.ds(g * _IROWS, _IROWS)], idxb)

        def gdesc(c, p, r):
            lrow = (c % _IGRP) * _NRI + r
            return pltpu.make_async_copy(
                k_hbm.at[idxb.at[lrow]],
                kgs[p].at[pl.ds(r * 128, 128)], gsems[p])

        def wdesc(c, p):
            return pltpu.make_async_copy(
                kgs[p], kg_hbm.at[bh, pl.ds(c * _ROWS, _ROWS)], wsems[p])

        def start_g(c, p):
            for r in range(_NRI):
                gdesc(c, p, r).start()

        def wait_g(c, p):
            for r in range(_NRI):
                gdesc(c, p, r).wait()

        stage_idx(0)
        start_g(0, 0)

        def two(j, carry):
            for p in (0, 1):
                c = j * 2 + p
                wait_g(c, p)
                wdesc(c, p).start()
                if p == 0:
                    @pl.when(c > 0)
                    def _():
                        wdesc(c - 1, 1).wait()

                    @pl.when(c + 1 < _NCH)
                    def _():
                        start_g(c + 1, 1)
                else:
                    wdesc(c - 1, 0).wait()

                    @pl.when(c + 1 < _NCH)
                    def _():
                        @pl.when((c + 1) % _IGRP == 0)
                        def _():
                            stage_idx((c + 1) // _IGRP)
                        start_g(c + 1, 0)
            return carry

        lax.fori_loop(0, _NCH // 2, two, 0)
        wdesc(_NCH - 1, 1).wait()

    return run(kflat, tab)


_QSEG = [(0, 2), (2, 5), (5, 8), (8, 10), (10, 13), (13, 16)]


def _sc_m_scores(kflat, q3, tab):
    mesh = plsc.VectorSubcoreMesh(core_axis_name="c", subcore_axis_name="s")
    out_type = (
        jax.ShapeDtypeStruct((_NBH, _L), jnp.float32),
        jax.ShapeDtypeStruct((_NBH, _L), jnp.float32),
    )
    scratch = [
        pltpu.VMEM((_ROWS, _D), jnp.float32),
        pltpu.VMEM((_NL * _IGRP, _D), jnp.float32),
        pltpu.VMEM((_IROWS, 128), jnp.int32),
        pltpu.VMEM((_L,), jnp.float32),
        pltpu.VMEM((_L,), jnp.float32),
    ] + [pltpu.SemaphoreType.DMA] * _NRI

    @functools.partial(pl.kernel, mesh=mesh, out_type=out_type,
                       scratch_types=scratch,
                       compiler_params=pltpu.CompilerParams(
                           use_tc_tiling_on_sc=False,
                           needs_layout_passes=False))
    def run(k_hbm, q_hbm, t_hbm, mx_hbm, ms_hbm,
            kg, qgb, idxb, mxr, msr, *sems):
        bh = lax.axis_index("s") * 2 + lax.axis_index("c")
        iota16 = lax.iota(jnp.int32, 16)

        def desc(c, r):
            lrow = (c % _IGRP) * _NRI + r
            return pltpu.make_async_copy(
                k_hbm.at[idxb.at[lrow]],
                kg.at[pl.ds(r * 128, 128)], sems[r])

        def gbody(g, carry):
            pltpu.sync_copy(t_hbm.at[bh, pl.ds(g * _IROWS, _IROWS)], idxb)
            pltpu.sync_copy(
                q_hbm.at[bh, pl.ds(g * (_NL * _IGRP), _NL * _IGRP)], qgb)

            def cbody(c, carry):
                for r in range(_NRI):
                    desc(c, r).start()
                qoff = (c % _IGRP) * _NL
                vmax = jnp.zeros((16,), jnp.float32)
                vsum = jnp.zeros((16,), jnp.float32)

                def lbody(i, carry):
                    vmax, vsum = carry
                    qrow = qoff + i
                    q0 = qgb[qrow, pl.ds(0, 16)]
                    q1 = qgb[qrow, pl.ds(16, 16)]
                    q2 = qgb[qrow, pl.ds(32, 16)]
                    q3v = qgb[qrow, pl.ds(48, 16)]
                    smax = jnp.float32(-3.0e38)
                    ssum = jnp.float32(0.0)
                    for s in range(_U):
                        row = i * _SP + s
                        a = ((kg[row, pl.ds(0, 16)] * q0
                              + kg[row, pl.ds(16, 16)] * q1)
                             + (kg[row, pl.ds(32, 16)] * q2
                                + kg[row, pl.ds(48, 16)] * q3v))
                        dot = plsc.cumsum(a)[15]
                        smax = jnp.maximum(smax, dot)
                        ssum = ssum + dot
                    lane = iota16 == i
                    return (jnp.where(lane, smax, vmax),
                            jnp.where(lane, ssum, vsum))

                for r, (lo, hi) in enumerate(_QSEG):
                    desc(c, r).wait()
                    vmax, vsum = lax.fori_loop(lo, hi, lbody, (vmax, vsum))
                mxr[pl.ds(c * _NL, _NL)] = vmax
                msr[pl.ds(c * _NL, _NL)] = vsum
                return carry

            lax.fori_loop(g * _IGRP, (g + 1) * _IGRP, cbody, 0)
            return carry

        lax.fori_loop(0, _NCH // _IGRP, gbody, 0)
        pltpu.sync_copy(mxr, mx_hbm.at[bh])
        pltpu.sync_copy(msr, ms_hbm.at[bh])

    return run(kflat, q3, tab)


def _tc_mreduce_body(kg_ref, q_ref, mx_ref, ms_ref):
    kg = kg_ref[0, 0].reshape(_BQ, _SP, _D)
    q = q_ref[0].reshape(_BQ, 1, _D)
    dots = jnp.sum(kg * q, axis=-1)
    svalid = lax.broadcasted_iota(jnp.int32, (_BQ, _SP), 1) < _U
    mx_ref[0, 0, 0] = jnp.max(jnp.where(svalid, dots, jnp.float32(-3.0e38)),
                              axis=-1)
    ms_ref[0, 0, 0] = jnp.sum(jnp.where(svalid, dots, jnp.float32(0.0)),
                              axis=-1)


def _tc_mreduce(kg, q3, interpret=False):
    mx4, ms4 = pl.pallas_call(
        _tc_mreduce_body,
        grid=(_NBH, _NQB),
        in_specs=[
            pl.BlockSpec((1, 1, _BQ * _SP, _D), lambda i, j: (i, j, 0, 0)),
            pl.BlockSpec((1, _BQ, _D), lambda i, j: (i, j, 0)),
        ],
        out_specs=[
            pl.BlockSpec((1, 1, 1, _BQ), lambda i, j: (i, j, 0, 0)),
            pl.BlockSpec((1, 1, 1, _BQ), lambda i, j: (i, j, 0, 0)),
        ],
        out_shape=[
            jax.ShapeDtypeStruct((_NBH, _NQB, 1, _BQ), jnp.float32),
            jax.ShapeDtypeStruct((_NBH, _NQB, 1, _BQ), jnp.float32),
        ],
        interpret=interpret,
    )(kg, q3)
    return mx4.reshape(_NBH, _L), ms4.reshape(_NBH, _L)


def _tc_topk_body(mx_ref, ms_ref, o_ref):
    M = mx_ref[...] - ms_ref[...] * jnp.float32(1.0 / _L)
    ci = lax.broadcasted_iota(jnp.int32, (_NBH, _L), 1).astype(jnp.float32)
    lane = lax.broadcasted_iota(jnp.int32, (_NBH, 128), 1).astype(jnp.float32)
    out = jnp.zeros((_NBH, 128), jnp.float32)
    X = M
    for i in range(_U):
        rmax = jnp.max(X, axis=1, keepdims=True)
        loc = jnp.min(jnp.where(X == rmax, ci, jnp.float32(1e9)),
                      axis=1, keepdims=True)
        out = jnp.where(lane == jnp.float32(i), loc, out)
        X = jnp.where(ci == loc, jnp.float32(-3.0e38), X)
    o_ref[...] = out.astype(jnp.int32)


def _tc_topk(mmax, msum, interpret=False):
    return pl.pallas_call(
        _tc_topk_body,
        out_shape=jax.ShapeDtypeStruct((_NBH, 128), jnp.int32),
        interpret=interpret,
    )(mmax, msum)


def _tc_attend_body(mt_ref, q_ref, k_ref, v_ref, o_ref, scr):
    for u in range(_U):
        r = mt_ref[0, 0, u]
        scr[pl.ds(u, 1), :] = q_ref[0, pl.ds(r, 1), :]
    scr[pl.ds(_U, 3), :] = jnp.zeros((3, _D), jnp.float32)
    qr = scr[...]
    k = k_ref[0]
    v = v_ref[0]
    S = lax.dot_general(qr, k, (((1,), (1,)), ((), ())),
                        preferred_element_type=jnp.float32)
    S = S * jnp.float32(1.0 / sqrt(_D))
    smx = jnp.max(S, axis=1, keepdims=True)
    E = jnp.exp(S - smx)
    P = E / jnp.sum(E, axis=1, keepdims=True)
    U = lax.dot_general(P, v, (((1,), (0,)), ((), ())),
                        preferred_element_type=jnp.float32)
    vmean = jnp.mean(v, axis=0, keepdims=True)
    o_ref[0] = jnp.broadcast_to(vmean, (_L, _D))
    for u in range(_U):
        r = mt_ref[0, 0, u]
        o_ref[0, pl.ds(r, 1), :] = U[u:u + 1, :]


def _tc_attend(mtop, q3, k3, v3, interpret=False):
    return pl.pallas_call(
        _tc_attend_body,
        grid=(_NBH,),
        in_specs=[
            pl.BlockSpec((1, 1, 128), lambda i: (i, 0, 0),
                         memory_space=pltpu.SMEM),
            pl.BlockSpec((1, _L, _D), lambda i: (i, 0, 0)),
            pl.BlockSpec((1, _L, _D), lambda i: (i, 0, 0)),
            pl.BlockSpec((1, _L, _D), lambda i: (i, 0, 0)),
        ],
        out_specs=pl.BlockSpec((1, _L, _D), lambda i: (i, 0, 0)),
        out_shape=jax.ShapeDtypeStruct((_NBH, _L, _D), jnp.float32),
        scratch_shapes=[pltpu.VMEM((_SP, _D), jnp.float32)],
        interpret=interpret,
    )(mtop.reshape(_NBH, 1, 128), q3, k3, v3)


def _tc_prep_body(k_ref, kf_ref):
    kf_ref[...] = k_ref[0]


def _tc_prep(k3, interpret=False):
    return pl.pallas_call(
        _tc_prep_body,
        grid=(_NBH,),
        in_specs=[pl.BlockSpec((1, _L, _D), lambda i: (i, 0, 0))],
        out_specs=pl.BlockSpec((_L, _D), lambda i: (i, 0)),
        out_shape=jax.ShapeDtypeStruct((_NBH * _L, _D), jnp.float32),
        interpret=interpret,
    )(k3)


def _sample_table():
    idx = jax.random.randint(jax.random.key(42), (_L, _U), 0, _L)
    idx48 = jnp.concatenate([idx, jnp.tile(idx[:, _U - 1:_U], (1, _SP - _U))],
                            axis=1)
    flat = idx48.reshape(-1).astype(jnp.int32)
    tab = flat[None, :] + (jnp.arange(_NBH, dtype=jnp.int32) * _L)[:, None]
    return tab.reshape(_NBH, (_L * _SP) // 128, 128)


def kernel(queries, keys, values):
    q3 = jnp.transpose(queries, (0, 2, 1, 3)).reshape(_NBH, _L, _D)
    k3 = jnp.transpose(keys, (0, 2, 1, 3)).reshape(_NBH, _L, _D)
    v3 = jnp.transpose(values, (0, 2, 1, 3)).reshape(_NBH, _L, _D)
    kflat = _tc_prep(k3)
    tab = _sample_table()
    mmax, msum = _sc_m_scores(kflat, q3, tab)
    mtop = _tc_topk(mmax, msum)
    ctx = _tc_attend(mtop, q3, k3, v3)
    return jnp.transpose(ctx.reshape(_B, _H, _L, _D), (0, 2, 1, 3))
